# initial kernel scaffold (unmeasured)
import functools

import jax
import jax.numpy as jnp
from jax import lax
from jax.experimental import pallas as pl
from jax.experimental.pallas import tpu as pltpu

N_DEV = 16
SQ = 256
SKV = 4096
H_PER = 8
DH = 128
D_MODEL = 1024
SCALE = 0.08838834764831843
BLK = 64


def _body(x_ref, wq_ref, k_ref, v_ref, wo_ref, out_ref,
          ctx_ref, comm_ref, send_sems, recv_sems):
    my = lax.axis_index("i")
    left = lax.rem(my + N_DEV - 1, N_DEV)
    right = lax.rem(my + 1, N_DEV)

    barrier_sem = pltpu.get_barrier_semaphore()
    for nbr in (left, right):
        pl.semaphore_signal(
            barrier_sem, inc=1,
            device_id=(nbr,), device_id_type=pl.DeviceIdType.MESH,
        )
    pl.semaphore_wait(barrier_sem, 2)

    q = jnp.dot(x_ref[:, :], wq_ref[:, :], preferred_element_type=jnp.float32)

    row = lax.broadcasted_iota(jnp.int32, (SQ, SKV), 0)
    col = lax.broadcasted_iota(jnp.int32, (SQ, SKV), 1)
    qb = row // BLK
    kb = col // BLK
    mask = (qb == kb) | (kb == 0) | (((qb + kb) % 3) == 0)

    for h in range(H_PER):
        qh = q[:, h * DH:(h + 1) * DH]
        kh = k_ref[:, h * DH:(h + 1) * DH]
        vh = v_ref[:, h * DH:(h + 1) * DH]
        scores = lax.dot_general(
            qh, kh, (((1,), (1,)), ((), ())),
            preferred_element_type=jnp.float32,
        ) * SCALE
        scores = jnp.where(mask, scores, -1e9)
        m = jnp.max(scores, axis=1, keepdims=True)
        e = jnp.exp(scores - m)
        s = jnp.sum(e, axis=1, keepdims=True)
        w = e / s
        ctx_ref[:, h * DH:(h + 1) * DH] = jnp.dot(
            w, vh, preferred_element_type=jnp.float32)

    partial = jnp.dot(ctx_ref[:, :], wo_ref[:, :],
                      preferred_element_type=jnp.float32)
    out_ref[:, :] = partial
    comm_ref[0, :, :] = partial

    for h in range(N_DEV - 1):
        rdma = pltpu.make_async_remote_copy(
            src_ref=comm_ref.at[h],
            dst_ref=comm_ref.at[h + 1],
            send_sem=send_sems.at[h],
            recv_sem=recv_sems.at[h],
            device_id=(right,),
            device_id_type=pl.DeviceIdType.MESH,
        )
        rdma.start()
        rdma.wait()
        out_ref[:, :] += comm_ref[h + 1, :, :]


def kernel(x, Wq, K_ext, V_ext, Wo):
    my = lax.axis_index("i")
    x2 = x.reshape(SQ, D_MODEL)
    k2 = K_ext.reshape(SKV, H_PER * DH)
    v2 = V_ext.reshape(SKV, H_PER * DH)
    wq_l = lax.dynamic_slice(Wq, (0, my * H_PER * DH), (D_MODEL, H_PER * DH))
    wo_l = lax.dynamic_slice(Wo, (my * H_PER * DH, 0), (H_PER * DH, D_MODEL))

    out = pl.pallas_call(
        _body,
        out_shape=jax.ShapeDtypeStruct((SQ, D_MODEL), jnp.float32),
        in_specs=[pl.BlockSpec(memory_space=pltpu.VMEM)] * 5,
        out_specs=pl.BlockSpec(memory_space=pltpu.VMEM),
        scratch_shapes=[
            pltpu.VMEM((SQ, H_PER * DH), jnp.float32),
            pltpu.VMEM((N_DEV, SQ, D_MODEL), jnp.float32),
            pltpu.SemaphoreType.DMA((N_DEV - 1,)),
            pltpu.SemaphoreType.DMA((N_DEV - 1,)),
        ],
        compiler_params=pltpu.CompilerParams(collective_id=0),
    )(x2, wq_l, k2, v2, wo_l)
    return out.reshape(1, SQ, D_MODEL)


# baseline (device time: 273447 ns/iter reference)
import jax
import jax.numpy as jnp
from jax import lax
from jax.experimental import pallas as pl
from jax.experimental.pallas import tpu as pltpu

N_DEV = 16
SQ = 256
SKV = 4096
H_PER = 8
DH = 128
D_MODEL = 1024
SCALE = 0.08838834764831843
BLK = 64


def _body(x_ref, wq_ref, k_hbm, v_hbm, wo_ref, out_ref,
          ctx_ref, comm_ref, kbuf, vbuf, ksems, vsems,
          send_sems, recv_sems):
    my = lax.axis_index("i")
    left = lax.rem(my + N_DEV - 1, N_DEV)
    right = lax.rem(my + 1, N_DEV)

    barrier_sem = pltpu.get_barrier_semaphore()
    for nbr in (left, right):
        pl.semaphore_signal(
            barrier_sem, inc=1,
            device_id=(nbr,), device_id_type=pl.DeviceIdType.MESH,
        )
    pl.semaphore_wait(barrier_sem, 2)

    def kv_copies(h, slot):
        kc = pltpu.make_async_copy(
            k_hbm.at[:, pl.ds(h * DH, DH)], kbuf.at[slot], ksems.at[slot])
        vc = pltpu.make_async_copy(
            v_hbm.at[:, pl.ds(h * DH, DH)], vbuf.at[slot], vsems.at[slot])
        return kc, vc

    kc0, vc0 = kv_copies(0, 0)
    kc0.start()
    vc0.start()

    q = jnp.dot(x_ref[:, :], wq_ref[:, :], preferred_element_type=jnp.float32)

    row = lax.broadcasted_iota(jnp.int32, (SQ, SKV), 0)
    col = lax.broadcasted_iota(jnp.int32, (SQ, SKV), 1)
    qb = row // BLK
    kb = col // BLK
    mask = (qb == kb) | (kb == 0) | (((qb + kb) % 3) == 0)

    for h in range(H_PER):
        slot = h % 2
        if h + 1 < H_PER:
            kc, vc = kv_copies(h + 1, (h + 1) % 2)
            kc.start()
            vc.start()
        kw, vw = kv_copies(h, slot)
        kw.wait()
        vw.wait()
        qh = q[:, h * DH:(h + 1) * DH]
        scores = lax.dot_general(
            qh, kbuf[slot], (((1,), (1,)), ((), ())),
            preferred_element_type=jnp.float32,
        ) * SCALE
        scores = jnp.where(mask, scores, -1e9)
        m = jnp.max(scores, axis=1, keepdims=True)
        e = jnp.exp(scores - m)
        s = jnp.sum(e, axis=1, keepdims=True)
        w = e / s
        ctx_ref[:, h * DH:(h + 1) * DH] = jnp.dot(
            w, vbuf[slot], preferred_element_type=jnp.float32)

    partial = jnp.dot(ctx_ref[:, :], wo_ref[:, :],
                      preferred_element_type=jnp.float32)
    out_ref[:, :] = partial
    comm_ref[0, :, :] = partial

    for h in range(N_DEV - 1):
        rdma = pltpu.make_async_remote_copy(
            src_ref=comm_ref.at[h],
            dst_ref=comm_ref.at[h + 1],
            send_sem=send_sems.at[h],
            recv_sem=recv_sems.at[h],
            device_id=(right,),
            device_id_type=pl.DeviceIdType.MESH,
        )
        rdma.start()
        rdma.wait()
        out_ref[:, :] += comm_ref[h + 1, :, :]


def kernel(x, Wq, K_ext, V_ext, Wo):
    my = lax.axis_index("i")
    x2 = x.reshape(SQ, D_MODEL)
    k2 = K_ext.reshape(SKV, H_PER * DH)
    v2 = V_ext.reshape(SKV, H_PER * DH)
    wq_l = lax.dynamic_slice(Wq, (0, my * H_PER * DH), (D_MODEL, H_PER * DH))
    wo_l = lax.dynamic_slice(Wo, (my * H_PER * DH, 0), (H_PER * DH, D_MODEL))

    out = pl.pallas_call(
        _body,
        out_shape=jax.ShapeDtypeStruct((SQ, D_MODEL), jnp.float32),
        in_specs=[
            pl.BlockSpec(memory_space=pltpu.VMEM),
            pl.BlockSpec(memory_space=pltpu.VMEM),
            pl.BlockSpec(memory_space=pl.ANY),
            pl.BlockSpec(memory_space=pl.ANY),
            pl.BlockSpec(memory_space=pltpu.VMEM),
        ],
        out_specs=pl.BlockSpec(memory_space=pltpu.VMEM),
        scratch_shapes=[
            pltpu.VMEM((SQ, H_PER * DH), jnp.float32),
            pltpu.VMEM((N_DEV, SQ, D_MODEL), jnp.float32),
            pltpu.VMEM((2, SKV, DH), jnp.float32),
            pltpu.VMEM((2, SKV, DH), jnp.float32),
            pltpu.SemaphoreType.DMA((2,)),
            pltpu.SemaphoreType.DMA((2,)),
            pltpu.SemaphoreType.DMA((N_DEV - 1,)),
            pltpu.SemaphoreType.DMA((N_DEV - 1,)),
        ],
        compiler_params=pltpu.CompilerParams(
            collective_id=0, vmem_limit_bytes=60 * 1024 * 1024),
    )(x2, wq_l, k2, v2, wo_l)
    return out.reshape(1, SQ, D_MODEL)


# device time: 120292 ns/iter; 2.2732x vs baseline; 2.2732x over previous
import jax
import jax.numpy as jnp
from jax import lax
from jax.experimental import pallas as pl
from jax.experimental.pallas import tpu as pltpu

N_DEV = 16
SQ = 256
SKV = 4096
H_PER = 8
DH = 128
D_MODEL = 1024
SCALE = 0.08838834764831843
BLK = 64


def _body(x_ref, wq_ref, k_hbm, v_hbm, wo_ref, out_ref,
          ctx_ref, acc_ref, a_land, b_land, kbuf, vbuf, ksems, vsems,
          send_sems, recv_sems):
    my = lax.axis_index("i")
    plane = my // 4
    pos = lax.rem(my, 4)
    p_right = plane * 4 + lax.rem(pos + 1, 4)
    p_left = plane * 4 + lax.rem(pos + 3, 4)
    z_next = lax.rem(plane + 1, 4) * 4 + pos
    z_prev = lax.rem(plane + 3, 4) * 4 + pos

    barrier_sem = pltpu.get_barrier_semaphore()
    for nbr in (p_left, p_right, z_prev, z_next):
        pl.semaphore_signal(
            barrier_sem, inc=1,
            device_id=(nbr,), device_id_type=pl.DeviceIdType.MESH,
        )
    pl.semaphore_wait(barrier_sem, 4)

    def kv_copies(h, slot):
        kc = pltpu.make_async_copy(
            k_hbm.at[:, pl.ds(h * DH, DH)], kbuf.at[slot], ksems.at[slot])
        vc = pltpu.make_async_copy(
            v_hbm.at[:, pl.ds(h * DH, DH)], vbuf.at[slot], vsems.at[slot])
        return kc, vc

    kc0, vc0 = kv_copies(0, 0)
    kc0.start()
    vc0.start()

    q = jnp.dot(x_ref[:, :], wq_ref[:, :], preferred_element_type=jnp.float32)

    row = lax.broadcasted_iota(jnp.int32, (SQ, SKV), 0)
    col = lax.broadcasted_iota(jnp.int32, (SQ, SKV), 1)
    qb = row // BLK
    kb = col // BLK
    mask = (qb == kb) | (kb == 0) | (((qb + kb) % 3) == 0)

    for h in range(H_PER):
        slot = h % 2
        if h + 1 < H_PER:
            kc, vc = kv_copies(h + 1, (h + 1) % 2)
            kc.start()
            vc.start()
        kw, vw = kv_copies(h, slot)
        kw.wait()
        vw.wait()
        qh = q[:, h * DH:(h + 1) * DH]
        scores = lax.dot_general(
            qh, kbuf[slot], (((1,), (1,)), ((), ())),
            preferred_element_type=jnp.float32,
        ) * SCALE
        scores = jnp.where(mask, scores, -1e9)
        m = jnp.max(scores, axis=1, keepdims=True)
        e = jnp.exp(scores - m)
        s = jnp.sum(e, axis=1, keepdims=True)
        w = e / s
        ctx_ref[:, h * DH:(h + 1) * DH] = jnp.dot(
            w, vbuf[slot], preferred_element_type=jnp.float32)

    acc_ref[:, :] = jnp.dot(ctx_ref[:, :], wo_ref[:, :],
                            preferred_element_type=jnp.float32)


    def hop(sem_idx, src, dst, target):
        rdma = pltpu.make_async_remote_copy(
            src_ref=src, dst_ref=dst,
            send_sem=send_sems.at[sem_idx],
            recv_sem=recv_sems.at[sem_idx],
            device_id=(target,),
            device_id_type=pl.DeviceIdType.MESH,
        )
        rdma.start()
        rdma.wait()

    for s in range(3):
        c_send = lax.rem(pos - s + 4, 4)
        c_recv = lax.rem(pos - s + 3, 4)
        hop(s,
            acc_ref.at[pl.ds(c_send * 64, 64), :],
            a_land.at[s],
            p_right)
        acc_ref[pl.ds(c_recv * 64, 64), :] += a_land[s]

    q_own = lax.rem(pos + 1, 4)
    row0 = q_own * 64

    for s in range(3):
        t_send = lax.rem(plane - s + 4, 4)
        t_recv = lax.rem(plane - s + 3, 4)
        hop(3 + s,
            acc_ref.at[pl.ds(row0 + t_send * 16, 16), :],
            b_land.at[s],
            z_next)
        acc_ref[pl.ds(row0 + t_recv * 16, 16), :] += b_land[s]
    for s in range(3):
        t_send = lax.rem(plane + 1 - s + 4, 4)
        hop(6 + s,
            acc_ref.at[pl.ds(row0 + t_send * 16, 16), :],
            acc_ref.at[pl.ds(row0 + t_send * 16, 16), :],
            z_next)

    for s in range(3):
        c_send = lax.rem(pos + 1 - s + 4, 4)
        hop(9 + s,
            acc_ref.at[pl.ds(c_send * 64, 64), :],
            acc_ref.at[pl.ds(c_send * 64, 64), :],
            p_right)

    out_ref[:, :] = acc_ref[:, :]


def kernel(x, Wq, K_ext, V_ext, Wo):
    my = lax.axis_index("i")
    x2 = x.reshape(SQ, D_MODEL)
    k2 = K_ext.reshape(SKV, H_PER * DH)
    v2 = V_ext.reshape(SKV, H_PER * DH)
    wq_l = lax.dynamic_slice(Wq, (0, my * H_PER * DH), (D_MODEL, H_PER * DH))
    wo_l = lax.dynamic_slice(Wo, (my * H_PER * DH, 0), (H_PER * DH, D_MODEL))

    out = pl.pallas_call(
        _body,
        out_shape=jax.ShapeDtypeStruct((SQ, D_MODEL), jnp.float32),
        in_specs=[
            pl.BlockSpec(memory_space=pltpu.VMEM),
            pl.BlockSpec(memory_space=pltpu.VMEM),
            pl.BlockSpec(memory_space=pl.ANY),
            pl.BlockSpec(memory_space=pl.ANY),
            pl.BlockSpec(memory_space=pltpu.VMEM),
        ],
        out_specs=pl.BlockSpec(memory_space=pltpu.VMEM),
        scratch_shapes=[
            pltpu.VMEM((SQ, H_PER * DH), jnp.float32),
            pltpu.VMEM((SQ, D_MODEL), jnp.float32),
            pltpu.VMEM((3, 64, D_MODEL), jnp.float32),
            pltpu.VMEM((3, 16, D_MODEL), jnp.float32),
            pltpu.VMEM((2, SKV, DH), jnp.float32),
            pltpu.VMEM((2, SKV, DH), jnp.float32),
            pltpu.SemaphoreType.DMA((2,)),
            pltpu.SemaphoreType.DMA((2,)),
            pltpu.SemaphoreType.DMA((12,)),
            pltpu.SemaphoreType.DMA((12,)),
        ],
        compiler_params=pltpu.CompilerParams(
            collective_id=0, vmem_limit_bytes=60 * 1024 * 1024),
    )(x2, wq_l, k2, v2, wo_l)
    return out.reshape(1, SQ, D_MODEL)


# device time: 113910 ns/iter; 2.4006x vs baseline; 1.0560x over previous
import jax
import jax.numpy as jnp
from jax import lax
from jax.experimental import pallas as pl
from jax.experimental.pallas import tpu as pltpu

N_DEV = 16
SQ = 256
SKV = 4096
H_PER = 8
DH = 128
D_MODEL = 1024
SCALE = 0.08838834764831843
BLK = 64


def _body(x_ref, wq_hbm, k_hbm, v_hbm, wo_hbm, out_ref,
          ctx_ref, acc_ref, a_land, b_land, kbuf, vbuf,
          wq_ref, wo_ref, ksems, vsems, wsems,
          send_sems, recv_sems):
    my = lax.axis_index("i")
    plane = my // 4
    pos = lax.rem(my, 4)
    p_right = plane * 4 + lax.rem(pos + 1, 4)
    p_left = plane * 4 + lax.rem(pos + 3, 4)
    z_next = lax.rem(plane + 1, 4) * 4 + pos
    z_prev = lax.rem(plane + 3, 4) * 4 + pos

    barrier_sem = pltpu.get_barrier_semaphore()
    for nbr in (p_left, p_right, z_prev, z_next):
        pl.semaphore_signal(
            barrier_sem, inc=1,
            device_id=(nbr,), device_id_type=pl.DeviceIdType.MESH,
        )
    pl.semaphore_wait(barrier_sem, 4)

    def kv_copies(h, slot):
        kc = pltpu.make_async_copy(
            k_hbm.at[:, pl.ds(h * DH, DH)], kbuf.at[slot], ksems.at[slot])
        vc = pltpu.make_async_copy(
            v_hbm.at[:, pl.ds(h * DH, DH)], vbuf.at[slot], vsems.at[slot])
        return kc, vc

    wq_cp = pltpu.make_async_copy(
        wq_hbm.at[:, pl.ds(my * H_PER * DH, H_PER * DH)], wq_ref, wsems.at[0])
    wo_cp = pltpu.make_async_copy(
        wo_hbm.at[pl.ds(my * H_PER * DH, H_PER * DH), :], wo_ref, wsems.at[1])
    wq_cp.start()
    wo_cp.start()

    kc0, vc0 = kv_copies(0, 0)
    kc0.start()
    vc0.start()

    wq_cp.wait()
    q = jnp.dot(x_ref[:, :], wq_ref[:, :], preferred_element_type=jnp.float32)

    row = lax.broadcasted_iota(jnp.int32, (SQ, SKV), 0)
    col = lax.broadcasted_iota(jnp.int32, (SQ, SKV), 1)
    qb = row // BLK
    kb = col // BLK
    mask = (qb == kb) | (kb == 0) | (((qb + kb) % 3) == 0)

    for h in range(H_PER):
        slot = h % 2
        if h + 1 < H_PER:
            kc, vc = kv_copies(h + 1, (h + 1) % 2)
            kc.start()
            vc.start()
        kw, vw = kv_copies(h, slot)
        kw.wait()
        vw.wait()
        qh = q[:, h * DH:(h + 1) * DH]
        scores = lax.dot_general(
            qh, kbuf[slot], (((1,), (1,)), ((), ())),
            preferred_element_type=jnp.float32,
        ) * SCALE
        scores = jnp.where(mask, scores, -1e9)
        m = jnp.max(scores, axis=1, keepdims=True)
        e = jnp.exp(scores - m)
        s = jnp.sum(e, axis=1, keepdims=True)
        w = e / s
        ctx_ref[:, h * DH:(h + 1) * DH] = jnp.dot(
            w, vbuf[slot], preferred_element_type=jnp.float32)

    wo_cp.wait()
    acc_ref[:, :] = jnp.dot(ctx_ref[:, :], wo_ref[:, :],
                            preferred_element_type=jnp.float32)


    def hop(sem_idx, src, dst, target):
        rdma = pltpu.make_async_remote_copy(
            src_ref=src, dst_ref=dst,
            send_sem=send_sems.at[sem_idx],
            recv_sem=recv_sems.at[sem_idx],
            device_id=(target,),
            device_id_type=pl.DeviceIdType.MESH,
        )
        rdma.start()
        rdma.wait()

    for s in range(3):
        c_send = lax.rem(pos - s + 4, 4)
        c_recv = lax.rem(pos - s + 3, 4)
        hop(s,
            acc_ref.at[pl.ds(c_send * 64, 64), :],
            a_land.at[s],
            p_right)
        acc_ref[pl.ds(c_recv * 64, 64), :] += a_land[s]

    q_own = lax.rem(pos + 1, 4)
    row0 = q_own * 64

    for s in range(3):
        t_send = lax.rem(plane - s + 4, 4)
        t_recv = lax.rem(plane - s + 3, 4)
        hop(3 + s,
            acc_ref.at[pl.ds(row0 + t_send * 16, 16), :],
            b_land.at[s],
            z_next)
        acc_ref[pl.ds(row0 + t_recv * 16, 16), :] += b_land[s]
    for s in range(3):
        t_send = lax.rem(plane + 1 - s + 4, 4)
        hop(6 + s,
            acc_ref.at[pl.ds(row0 + t_send * 16, 16), :],
            acc_ref.at[pl.ds(row0 + t_send * 16, 16), :],
            z_next)

    for s in range(3):
        c_send = lax.rem(pos + 1 - s + 4, 4)
        hop(9 + s,
            acc_ref.at[pl.ds(c_send * 64, 64), :],
            acc_ref.at[pl.ds(c_send * 64, 64), :],
            p_right)

    out_ref[:, :] = acc_ref[:, :]


def kernel(x, Wq, K_ext, V_ext, Wo):
    x2 = x.reshape(SQ, D_MODEL)
    k2 = K_ext.reshape(SKV, H_PER * DH)
    v2 = V_ext.reshape(SKV, H_PER * DH)

    out = pl.pallas_call(
        _body,
        out_shape=jax.ShapeDtypeStruct((SQ, D_MODEL), jnp.float32),
        in_specs=[
            pl.BlockSpec(memory_space=pltpu.VMEM),
            pl.BlockSpec(memory_space=pl.ANY),
            pl.BlockSpec(memory_space=pl.ANY),
            pl.BlockSpec(memory_space=pl.ANY),
            pl.BlockSpec(memory_space=pl.ANY),
        ],
        out_specs=pl.BlockSpec(memory_space=pltpu.VMEM),
        scratch_shapes=[
            pltpu.VMEM((SQ, H_PER * DH), jnp.float32),
            pltpu.VMEM((SQ, D_MODEL), jnp.float32),
            pltpu.VMEM((3, 64, D_MODEL), jnp.float32),
            pltpu.VMEM((3, 16, D_MODEL), jnp.float32),
            pltpu.VMEM((2, SKV, DH), jnp.float32),
            pltpu.VMEM((2, SKV, DH), jnp.float32),
            pltpu.VMEM((D_MODEL, H_PER * DH), jnp.float32),
            pltpu.VMEM((H_PER * DH, D_MODEL), jnp.float32),
            pltpu.SemaphoreType.DMA((2,)),
            pltpu.SemaphoreType.DMA((2,)),
            pltpu.SemaphoreType.DMA((2,)),
            pltpu.SemaphoreType.DMA((12,)),
            pltpu.SemaphoreType.DMA((12,)),
        ],
        compiler_params=pltpu.CompilerParams(
            collective_id=0, vmem_limit_bytes=60 * 1024 * 1024),
    )(x2, Wq, k2, v2, Wo)
    return out.reshape(1, SQ, D_MODEL)


# device time: 87569 ns/iter; 3.1226x vs baseline; 1.3008x over previous
import jax
import jax.numpy as jnp
from jax import lax
from jax.experimental import pallas as pl
from jax.experimental.pallas import tpu as pltpu

N_DEV = 16
SQ = 256
SKV = 4096
H_PER = 8
DH = 128
D_MODEL = 1024
SCALE = 0.08838834764831843
BLK = 64


def _body(x_ref, wq_hbm, k_hbm, v_hbm, wo_hbm, out_ref,
          ctx_ref, acc_ref, a_land, b_land, kbuf, vbuf,
          wq_ref, wo_ref, ksems, vsems, wsems,
          send_sems, recv_sems):
    my = lax.axis_index("i")
    plane = my // 4
    pos = lax.rem(my, 4)
    p_right = plane * 4 + lax.rem(pos + 1, 4)
    p_left = plane * 4 + lax.rem(pos + 3, 4)
    z_next = lax.rem(plane + 1, 4) * 4 + pos
    z_prev = lax.rem(plane + 3, 4) * 4 + pos

    barrier_sem = pltpu.get_barrier_semaphore()
    for nbr in (p_left, p_right, z_prev, z_next):
        pl.semaphore_signal(
            barrier_sem, inc=1,
            device_id=(nbr,), device_id_type=pl.DeviceIdType.MESH,
        )
    pl.semaphore_wait(barrier_sem, 4)

    def kv_copies(h, slot):
        kc = pltpu.make_async_copy(
            k_hbm.at[:, h, :], kbuf.at[slot], ksems.at[slot])
        vc = pltpu.make_async_copy(
            v_hbm.at[:, h, :], vbuf.at[slot], vsems.at[slot])
        return kc, vc

    wq_cp = pltpu.make_async_copy(
        wq_hbm.at[:, pl.ds(my * H_PER * DH, H_PER * DH)], wq_ref, wsems.at[0])
    wo_cp = pltpu.make_async_copy(
        wo_hbm.at[pl.ds(my * H_PER * DH, H_PER * DH), :], wo_ref, wsems.at[1])
    wq_cp.start()
    wo_cp.start()

    kc0, vc0 = kv_copies(0, 0)
    kc0.start()
    vc0.start()

    wq_cp.wait()
    q = jnp.dot(x_ref[:, :], wq_ref[:, :], preferred_element_type=jnp.float32)

    row = lax.broadcasted_iota(jnp.int32, (SQ, SKV), 0)
    col = lax.broadcasted_iota(jnp.int32, (SQ, SKV), 1)
    qb = row // BLK
    kb = col // BLK
    mask = (qb == kb) | (kb == 0) | (((qb + kb) % 3) == 0)

    for h in range(H_PER):
        slot = h % 2
        if h + 1 < H_PER:
            kc, vc = kv_copies(h + 1, (h + 1) % 2)
            kc.start()
            vc.start()
        kw, vw = kv_copies(h, slot)
        kw.wait()
        vw.wait()
        qh = q[:, h * DH:(h + 1) * DH]
        scores = lax.dot_general(
            qh, kbuf[slot], (((1,), (1,)), ((), ())),
            preferred_element_type=jnp.float32,
        ) * SCALE
        scores = jnp.where(mask, scores, -1e9)
        m = jnp.max(scores, axis=1, keepdims=True)
        e = jnp.exp(scores - m)
        s = jnp.sum(e, axis=1, keepdims=True)
        w = e / s
        ctx_ref[:, h * DH:(h + 1) * DH] = jnp.dot(
            w, vbuf[slot], preferred_element_type=jnp.float32)

    wo_cp.wait()
    acc_ref[:, :] = jnp.dot(ctx_ref[:, :], wo_ref[:, :],
                            preferred_element_type=jnp.float32)


    def hop(sem_idx, src, dst, target):
        rdma = pltpu.make_async_remote_copy(
            src_ref=src, dst_ref=dst,
            send_sem=send_sems.at[sem_idx],
            recv_sem=recv_sems.at[sem_idx],
            device_id=(target,),
            device_id_type=pl.DeviceIdType.MESH,
        )
        rdma.start()
        rdma.wait()

    for s in range(3):
        c_send = lax.rem(pos - s + 4, 4)
        c_recv = lax.rem(pos - s + 3, 4)
        hop(s,
            acc_ref.at[pl.ds(c_send * 64, 64), :],
            a_land.at[s],
            p_right)
        acc_ref[pl.ds(c_recv * 64, 64), :] += a_land[s]

    q_own = lax.rem(pos + 1, 4)
    row0 = q_own * 64

    for s in range(3):
        t_send = lax.rem(plane - s + 4, 4)
        t_recv = lax.rem(plane - s + 3, 4)
        hop(3 + s,
            acc_ref.at[pl.ds(row0 + t_send * 16, 16), :],
            b_land.at[s],
            z_next)
        acc_ref[pl.ds(row0 + t_recv * 16, 16), :] += b_land[s]
    for s in range(3):
        t_send = lax.rem(plane + 1 - s + 4, 4)
        hop(6 + s,
            acc_ref.at[pl.ds(row0 + t_send * 16, 16), :],
            acc_ref.at[pl.ds(row0 + t_send * 16, 16), :],
            z_next)

    for s in range(3):
        c_send = lax.rem(pos + 1 - s + 4, 4)
        hop(9 + s,
            acc_ref.at[pl.ds(c_send * 64, 64), :],
            acc_ref.at[pl.ds(c_send * 64, 64), :],
            p_right)

    out_ref[:, :] = acc_ref[:, :]


def kernel(x, Wq, K_ext, V_ext, Wo):
    x2 = x.reshape(SQ, D_MODEL)
    k2 = K_ext.reshape(SKV, H_PER, DH)
    v2 = V_ext.reshape(SKV, H_PER, DH)

    out = pl.pallas_call(
        _body,
        out_shape=jax.ShapeDtypeStruct((SQ, D_MODEL), jnp.float32),
        in_specs=[
            pl.BlockSpec(memory_space=pltpu.VMEM),
            pl.BlockSpec(memory_space=pl.ANY),
            pl.BlockSpec(memory_space=pl.ANY),
            pl.BlockSpec(memory_space=pl.ANY),
            pl.BlockSpec(memory_space=pl.ANY),
        ],
        out_specs=pl.BlockSpec(memory_space=pltpu.VMEM),
        scratch_shapes=[
            pltpu.VMEM((SQ, H_PER * DH), jnp.float32),
            pltpu.VMEM((SQ, D_MODEL), jnp.float32),
            pltpu.VMEM((3, 64, D_MODEL), jnp.float32),
            pltpu.VMEM((3, 16, D_MODEL), jnp.float32),
            pltpu.VMEM((2, SKV, DH), jnp.float32),
            pltpu.VMEM((2, SKV, DH), jnp.float32),
            pltpu.VMEM((D_MODEL, H_PER * DH), jnp.float32),
            pltpu.VMEM((H_PER * DH, D_MODEL), jnp.float32),
            pltpu.SemaphoreType.DMA((2,)),
            pltpu.SemaphoreType.DMA((2,)),
            pltpu.SemaphoreType.DMA((2,)),
            pltpu.SemaphoreType.DMA((12,)),
            pltpu.SemaphoreType.DMA((12,)),
        ],
        compiler_params=pltpu.CompilerParams(
            collective_id=0, vmem_limit_bytes=60 * 1024 * 1024),
    )(x2, Wq, k2, v2, Wo)
    return out.reshape(1, SQ, D_MODEL)


# device time: 79793 ns/iter; 3.4270x vs baseline; 1.0975x over previous
import jax
import jax.numpy as jnp
from jax import lax
from jax.experimental import pallas as pl
from jax.experimental.pallas import tpu as pltpu

N_DEV = 16
SQ = 256
SKV = 4096
H_PER = 8
DH = 128
D_MODEL = 1024
SCALE = 0.08838834764831843
BLK = 64


def _body(x_ref, wq_hbm, k_hbm, v_hbm, wo_hbm, out_ref,
          ctx_ref, acc_ref, a_land, b_land, kbuf, vbuf,
          wq_ref, wo_ref, ksems, vsems, wsems,
          send_sems, recv_sems):
    my = lax.axis_index("i")
    plane = my // 4
    pos = lax.rem(my, 4)
    p_right = plane * 4 + lax.rem(pos + 1, 4)
    p_left = plane * 4 + lax.rem(pos + 3, 4)
    z_next = lax.rem(plane + 1, 4) * 4 + pos
    z_prev = lax.rem(plane + 3, 4) * 4 + pos

    barrier_sem = pltpu.get_barrier_semaphore()
    for nbr in (p_left, p_right, z_prev, z_next):
        pl.semaphore_signal(
            barrier_sem, inc=1,
            device_id=(nbr,), device_id_type=pl.DeviceIdType.MESH,
        )
    pl.semaphore_wait(barrier_sem, 4)

    def kv_copies(h, slot):
        kc = pltpu.make_async_copy(
            k_hbm.at[:, h, :], kbuf.at[slot], ksems.at[slot])
        vc = pltpu.make_async_copy(
            v_hbm.at[:, h, :], vbuf.at[slot], vsems.at[slot])
        return kc, vc

    wq_cp = pltpu.make_async_copy(
        wq_hbm.at[:, pl.ds(my * H_PER * DH, H_PER * DH)], wq_ref, wsems.at[0])
    wo_cp = pltpu.make_async_copy(
        wo_hbm.at[pl.ds(my * H_PER * DH, H_PER * DH), :], wo_ref, wsems.at[1])
    wq_cp.start()
    wo_cp.start()

    kc0, vc0 = kv_copies(0, 0)
    kc0.start()
    vc0.start()

    bf16 = jnp.bfloat16
    wq_cp.wait()
    q = jnp.dot(x_ref[:, :].astype(bf16), wq_ref[:, :].astype(bf16),
                preferred_element_type=jnp.float32)

    row = lax.broadcasted_iota(jnp.int32, (SQ, SKV), 0)
    col = lax.broadcasted_iota(jnp.int32, (SQ, SKV), 1)
    qb = row // BLK
    kb = col // BLK
    mask = (qb == kb) | (kb == 0) | (((qb + kb) % 3) == 0)

    for h in range(H_PER):
        slot = h % 2
        if h + 1 < H_PER:
            kc, vc = kv_copies(h + 1, (h + 1) % 2)
            kc.start()
            vc.start()
        kw, vw = kv_copies(h, slot)
        kw.wait()
        vw.wait()
        qh = q[:, h * DH:(h + 1) * DH].astype(bf16)
        scores = lax.dot_general(
            qh, kbuf[slot].astype(bf16), (((1,), (1,)), ((), ())),
            preferred_element_type=jnp.float32,
        ) * SCALE
        scores = jnp.where(mask, scores, -1e9)
        m = jnp.max(scores, axis=1, keepdims=True)
        e = jnp.exp(scores - m)
        s = jnp.sum(e, axis=1, keepdims=True)
        w = (e / s).astype(bf16)
        ctx_ref[:, h * DH:(h + 1) * DH] = jnp.dot(
            w, vbuf[slot].astype(bf16), preferred_element_type=jnp.float32)

    wo_cp.wait()
    acc_ref[:, :] = jnp.dot(ctx_ref[:, :].astype(bf16),
                            wo_ref[:, :].astype(bf16),
                            preferred_element_type=jnp.float32)


    def hop(sem_idx, src, dst, target):
        rdma = pltpu.make_async_remote_copy(
            src_ref=src, dst_ref=dst,
            send_sem=send_sems.at[sem_idx],
            recv_sem=recv_sems.at[sem_idx],
            device_id=(target,),
            device_id_type=pl.DeviceIdType.MESH,
        )
        rdma.start()
        rdma.wait()

    for s in range(3):
        c_send = lax.rem(pos - s + 4, 4)
        c_recv = lax.rem(pos - s + 3, 4)
        hop(s,
            acc_ref.at[pl.ds(c_send * 64, 64), :],
            a_land.at[s],
            p_right)
        acc_ref[pl.ds(c_recv * 64, 64), :] += a_land[s]

    q_own = lax.rem(pos + 1, 4)
    row0 = q_own * 64

    for s in range(3):
        t_send = lax.rem(plane - s + 4, 4)
        t_recv = lax.rem(plane - s + 3, 4)
        hop(3 + s,
            acc_ref.at[pl.ds(row0 + t_send * 16, 16), :],
            b_land.at[s],
            z_next)
        acc_ref[pl.ds(row0 + t_recv * 16, 16), :] += b_land[s]
    for s in range(3):
        t_send = lax.rem(plane + 1 - s + 4, 4)
        hop(6 + s,
            acc_ref.at[pl.ds(row0 + t_send * 16, 16), :],
            acc_ref.at[pl.ds(row0 + t_send * 16, 16), :],
            z_next)

    for s in range(3):
        c_send = lax.rem(pos + 1 - s + 4, 4)
        hop(9 + s,
            acc_ref.at[pl.ds(c_send * 64, 64), :],
            acc_ref.at[pl.ds(c_send * 64, 64), :],
            p_right)

    out_ref[:, :] = acc_ref[:, :]


def kernel(x, Wq, K_ext, V_ext, Wo):
    x2 = x.reshape(SQ, D_MODEL)
    k2 = K_ext.reshape(SKV, H_PER, DH)
    v2 = V_ext.reshape(SKV, H_PER, DH)

    out = pl.pallas_call(
        _body,
        out_shape=jax.ShapeDtypeStruct((SQ, D_MODEL), jnp.float32),
        in_specs=[
            pl.BlockSpec(memory_space=pltpu.VMEM),
            pl.BlockSpec(memory_space=pl.ANY),
            pl.BlockSpec(memory_space=pl.ANY),
            pl.BlockSpec(memory_space=pl.ANY),
            pl.BlockSpec(memory_space=pl.ANY),
        ],
        out_specs=pl.BlockSpec(memory_space=pltpu.VMEM),
        scratch_shapes=[
            pltpu.VMEM((SQ, H_PER * DH), jnp.float32),
            pltpu.VMEM((SQ, D_MODEL), jnp.float32),
            pltpu.VMEM((3, 64, D_MODEL), jnp.float32),
            pltpu.VMEM((3, 16, D_MODEL), jnp.float32),
            pltpu.VMEM((2, SKV, DH), jnp.float32),
            pltpu.VMEM((2, SKV, DH), jnp.float32),
            pltpu.VMEM((D_MODEL, H_PER * DH), jnp.float32),
            pltpu.VMEM((H_PER * DH, D_MODEL), jnp.float32),
            pltpu.SemaphoreType.DMA((2,)),
            pltpu.SemaphoreType.DMA((2,)),
            pltpu.SemaphoreType.DMA((2,)),
            pltpu.SemaphoreType.DMA((12,)),
            pltpu.SemaphoreType.DMA((12,)),
        ],
        compiler_params=pltpu.CompilerParams(
            collective_id=0, vmem_limit_bytes=60 * 1024 * 1024),
    )(x2, Wq, k2, v2, Wo)
    return out.reshape(1, SQ, D_MODEL)
